# Initial kernel scaffold; baseline (speedup 1.0000x reference)
import jax
import jax.numpy as jnp
from jax.experimental import pallas as pl


def kernel(x, edge_index, W_l, W_r, att, bias):
    raise NotImplementedError("write your pallas kernel here")


# speedup vs baseline: 1.0007x; 1.0007x over previous
import jax
import jax.numpy as jnp
from jax.experimental import pallas as pl
from jax.experimental.pallas import tpu as pltpu

H = 8
C = 128
NEG_SLOPE = 0.2


def _mm_body(x_ref, wl_ref, wr_ref, xl_ref, xr_ref):
    x = x_ref[...]
    xl_ref[...] = jnp.dot(x, wl_ref[...], preferred_element_type=jnp.float32)
    xr_ref[...] = jnp.dot(x, wr_ref[...], preferred_element_type=jnp.float32)


def _project(x, W_l, W_r):
    N, d = x.shape
    HC = W_l.shape[1]
    BN = 400
    grid = (N // BN,)
    return pl.pallas_call(
        _mm_body,
        grid=grid,
        in_specs=[
            pl.BlockSpec((BN, d), lambda i: (i, 0)),
            pl.BlockSpec((d, HC), lambda i: (0, 0)),
            pl.BlockSpec((d, HC), lambda i: (0, 0)),
        ],
        out_specs=[
            pl.BlockSpec((BN, HC), lambda i: (i, 0)),
            pl.BlockSpec((BN, HC), lambda i: (i, 0)),
        ],
        out_shape=[
            jax.ShapeDtypeStruct((N, HC), jnp.float32),
            jax.ShapeDtypeStruct((N, HC), jnp.float32),
        ],
    )(x, W_l, W_r)


def kernel(x, edge_index, W_l, W_r, att, bias):
    N = x.shape[0]
    src = edge_index[0].astype(jnp.int32)
    dst = edge_index[1].astype(jnp.int32)
    xl2, xr2 = _project(x, W_l, W_r)
    x_l = xl2.reshape(N, H, C)
    x_r = xr2.reshape(N, H, C)
    e = x_l[src] + x_r[dst]
    e = jax.nn.leaky_relu(e, NEG_SLOPE)
    alpha = (e * att[None, :, :]).sum(-1)
    amax = jax.ops.segment_max(alpha, dst, num_segments=N)
    alpha = jnp.exp(alpha - amax[dst])
    asum = jax.ops.segment_sum(alpha, dst, num_segments=N)
    alpha = alpha / (asum[dst] + 1e-16)
    msg = x_l[src] * alpha[:, :, None]
    out = jax.ops.segment_sum(msg, dst, num_segments=N)
    return out.reshape(N, H * C) + bias


# speedup vs baseline: 3.1025x; 3.1002x over previous
import functools

import jax
import jax.numpy as jnp
from jax import lax
from jax.experimental import pallas as pl
from jax.experimental.pallas import tpu as pltpu
from jax.experimental.pallas import tpu_sc as plsc

H = 8
C = 128
HC = H * C
NEG_SLOPE = 0.2

N_NODES = 10000
N_EDGES = 320000

NW = 32
R = 82
PASSES = 4
CH = 1280
NCHUNKS = N_EDGES // CH
GPC = CH // 16


def _mm_body(x_ref, wl_ref, wr_ref, xl_ref, xr_ref):
    x = x_ref[...]
    xl_ref[...] = jnp.dot(x, wl_ref[...], preferred_element_type=jnp.float32)
    xr_ref[...] = jnp.dot(x, wr_ref[...], preferred_element_type=jnp.float32)


def _project(x, W_l, W_r):
    N, d = x.shape
    BN = 400
    return pl.pallas_call(
        _mm_body,
        grid=(N // BN,),
        in_specs=[
            pl.BlockSpec((BN, d), lambda i: (i, 0)),
            pl.BlockSpec((d, HC), lambda i: (0, 0)),
            pl.BlockSpec((d, HC), lambda i: (0, 0)),
        ],
        out_specs=[
            pl.BlockSpec((BN, HC), lambda i: (i, 0)),
            pl.BlockSpec((BN, HC), lambda i: (i, 0)),
        ],
        out_shape=[
            jax.ShapeDtypeStruct((N, HC), jnp.float32),
            jax.ShapeDtypeStruct((N, HC), jnp.float32),
        ],
    )(x, W_l, W_r)


def _sc_gat(xl, xr, src, dst, att1d, bias):
    mesh = plsc.VectorSubcoreMesh(core_axis_name="c", subcore_axis_name="s")

    @functools.partial(
        pl.kernel,
        out_type=jax.ShapeDtypeStruct((N_NODES, HC), jnp.float32),
        mesh=mesh,
        compiler_params=pltpu.CompilerParams(needs_layout_passes=False),
        scratch_types=[
            pltpu.VMEM((CH,), jnp.int32),
            pltpu.VMEM((CH,), jnp.int32),
            pltpu.VMEM((CH + 16,), jnp.int32),
            pltpu.VMEM((CH + 16,), jnp.int32),
            pltpu.VMEM((16, HC), jnp.float32),
            pltpu.VMEM((16, HC), jnp.float32),
            pltpu.VMEM((16, 16), jnp.float32),
            pltpu.VMEM((HC,), jnp.float32),
            pltpu.VMEM((HC,), jnp.float32),
            pltpu.VMEM(((R + 1) * HC,), jnp.float32),
            pltpu.VMEM(((R + 1) * 16,), jnp.float32),
            pltpu.SemaphoreType.DMA,
            pltpu.SemaphoreType.DMA,
        ],
    )
    def k(xl_hbm, xr_hbm, src_hbm, dst_hbm, att_hbm, bias_hbm, out_hbm,
          src_chunk, dst_chunk, csrc, cdst, xl_buf, xr_buf, p_e,
          att_buf, bias_buf, acc, psum, sem0, sem1):
        cid = lax.axis_index("c")
        sid = lax.axis_index("s")
        w = sid * 2 + cid

        i16 = lax.iota(jnp.int32, 16)
        lane0 = i16 == 0
        zf16 = jnp.zeros((16,), jnp.float32)

        pltpu.sync_copy(att_hbm, att_buf)
        pltpu.sync_copy(bias_hbm, bias_buf)

        def pass_body(p, _):
            lo = p * (NW * R) + w * R
            hi = jnp.minimum(lo + R, N_NODES)

            def zacc(i, _):
                acc[pl.ds(i * 16, 16)] = zf16
                return 0
            lax.fori_loop(0, (R + 1) * HC // 16, zacc, 0)

            def zps(i, _):
                psum[pl.ds(i * 16, 16)] = zf16
                return 0
            lax.fori_loop(0, R + 1, zps, 0)

            def chunk_body(ck, _):
                eoff = ck * CH
                cpa = pltpu.async_copy(
                    src_hbm.at[pl.ds(eoff, CH)], src_chunk, sem0)
                cpb = pltpu.async_copy(
                    dst_hbm.at[pl.ds(eoff, CH)], dst_chunk, sem1)
                cpa.wait()
                cpb.wait()

                def compact(g, cur):
                    d16 = dst_chunk[pl.ds(g * 16, 16)]
                    s16 = src_chunk[pl.ds(g * 16, 16)]
                    m = (d16 >= lo) & (d16 < hi)
                    m32 = m.astype(jnp.int32)
                    cs = plsc.cumsum(m32)
                    pos = cur + cs - 1
                    plsc.store_scatter(cdst, [pos], d16 - lo, mask=m)
                    plsc.store_scatter(csrc, [pos], s16, mask=m)
                    return cur + cs[15]
                nsel = lax.fori_loop(0, GPC, compact, 0)

                plsc.store_scatter(cdst, [nsel + i16],
                                   jnp.full((16,), R, jnp.int32))
                plsc.store_scatter(csrc, [nsel + i16],
                                   jnp.zeros((16,), jnp.int32))
                ngroups = (nsel + 15) // 16

                def pg(g2, _):
                    o = g2 * 16
                    cs16 = csrc[pl.ds(o, 16)]
                    cd16 = cdst[pl.ds(o, 16)]
                    dg = jnp.minimum(cd16 + lo, N_NODES - 1)
                    cpc = pltpu.async_copy(xl_hbm.at[cs16], xl_buf, sem0)
                    cpd = pltpu.async_copy(xr_hbm.at[dg], xr_buf, sem1)
                    cpc.wait()
                    cpd.wait()

                    for h in range(H):
                        attv = [att_buf[pl.ds(h * C + c * 16, 16)]
                                for c in range(8)]

                        def ebody(j, _, h=h, attv=attv):
                            a16 = zf16
                            for c in range(8):
                                off = h * C + c * 16
                                z = (xl_buf[j, pl.ds(off, 16)]
                                     + xr_buf[j, pl.ds(off, 16)])
                                zl = jnp.maximum(z, NEG_SLOPE * z)
                                a16 = a16 + attv[c] * zl
                            a = plsc.cumsum(a16)[15]
                            pv = jnp.exp(jnp.full((16,), a, jnp.float32))
                            plsc.store_scatter(
                                p_e,
                                [jnp.full((16,), j, jnp.int32),
                                 jnp.full((16,), h, jnp.int32)],
                                pv, mask=lane0)
                            return 0
                        lax.fori_loop(0, 16, ebody, 0)

                    def pedge(j, _):
                        djv = plsc.load_gather(
                            cdst, [jnp.full((16,), o, jnp.int32) + j])
                        dj = djv[0]
                        rb = dj * HC
                        for h in range(H):
                            pb = plsc.load_gather(
                                p_e,
                                [jnp.full((16,), j, jnp.int32),
                                 jnp.full((16,), h, jnp.int32)])
                            for c in range(8):
                                off = h * C + c * 16
                                v = xl_buf[j, pl.ds(off, 16)] * pb
                                plsc.addupdate(acc.at[pl.ds(rb + off, 16)], v)
                        prow = p_e[j, pl.ds(0, 16)]
                        plsc.addupdate(psum.at[pl.ds(dj * 16, 16)], prow)
                        return 0
                    lax.fori_loop(0, 16, pedge, 0)
                    return 0
                lax.fori_loop(0, ngroups, pg, 0)
                return 0
            lax.fori_loop(0, NCHUNKS, chunk_body, 0)

            def fin_row(r, _):
                n = lo + r

                @pl.when(n < N_NODES)
                def _():
                    pv = psum[pl.ds(r * 16, 16)]
                    psum[pl.ds(r * 16, 16)] = 1.0 / (pv + 1e-16)
                    for h in range(H):
                        ib = plsc.load_gather(
                            psum, [jnp.full((16,), r * 16 + h, jnp.int32)])
                        for c in range(8):
                            off = h * C + c * 16
                            xl_buf[0, pl.ds(off, 16)] = (
                                acc[pl.ds(r * HC + off, 16)] * ib
                                + bias_buf[pl.ds(off, 16)])
                    pltpu.sync_copy(xl_buf.at[0], out_hbm.at[n])
                return 0
            lax.fori_loop(0, R, fin_row, 0)
            return 0
        lax.fori_loop(0, PASSES, pass_body, 0)

    return k(xl, xr, src, dst, att1d, bias)


def kernel(x, edge_index, W_l, W_r, att, bias):
    src = edge_index[0].astype(jnp.int32)
    dst = edge_index[1].astype(jnp.int32)
    xl2, xr2 = _project(x, W_l, W_r)
    return _sc_gat(xl2, xr2, src, dst, att.reshape(-1),
                   bias.astype(jnp.float32))


# speedup vs baseline: 4.2133x; 1.3580x over previous
import functools

import jax
import jax.numpy as jnp
from jax import lax
from jax.experimental import pallas as pl
from jax.experimental.pallas import tpu as pltpu
from jax.experimental.pallas import tpu_sc as plsc

H = 8
C = 128
HC = H * C
NEG_SLOPE = 0.2

N_NODES = 10000
N_EDGES = 320000

NW = 32
R = 80
PASSES = 4
CH = 1280
NCHUNKS = N_EDGES // CH
GPC = CH // 16


def _mm_body(x_ref, wl_ref, wr_ref, xl_ref, xr_ref):
    x = x_ref[...]
    xl_ref[...] = jnp.dot(x, wl_ref[...], preferred_element_type=jnp.float32)
    xr_ref[...] = jnp.dot(x, wr_ref[...], preferred_element_type=jnp.float32)


def _project(x, W_l, W_r):
    N, d = x.shape
    BN = 400
    return pl.pallas_call(
        _mm_body,
        grid=(N // BN,),
        in_specs=[
            pl.BlockSpec((BN, d), lambda i: (i, 0)),
            pl.BlockSpec((d, HC), lambda i: (0, 0)),
            pl.BlockSpec((d, HC), lambda i: (0, 0)),
        ],
        out_specs=[
            pl.BlockSpec((BN, HC), lambda i: (i, 0)),
            pl.BlockSpec((BN, HC), lambda i: (i, 0)),
        ],
        out_shape=[
            jax.ShapeDtypeStruct((N, HC), jnp.float32),
            jax.ShapeDtypeStruct((N, HC), jnp.float32),
        ],
    )(x, W_l, W_r)


def _sc_gat(xl, xr, src, dst, att1d, bias):
    mesh = plsc.VectorSubcoreMesh(core_axis_name="c", subcore_axis_name="s")

    @functools.partial(
        pl.kernel,
        out_type=jax.ShapeDtypeStruct((N_NODES, HC), jnp.float32),
        mesh=mesh,
        compiler_params=pltpu.CompilerParams(needs_layout_passes=False),
        scratch_types=[
            pltpu.VMEM((CH,), jnp.int32),
            pltpu.VMEM((CH,), jnp.int32),
            pltpu.VMEM((CH + 16,), jnp.int32),
            pltpu.VMEM((CH + 16,), jnp.int32),
            pltpu.VMEM((16, HC), jnp.float32),
            pltpu.VMEM((16, HC), jnp.float32),
            pltpu.VMEM((HC,), jnp.float32),
            pltpu.VMEM((HC,), jnp.float32),
            pltpu.VMEM(((R + 1) * HC,), jnp.float32),
            pltpu.VMEM(((R + 1) * 16,), jnp.float32),
            pltpu.SemaphoreType.DMA,
            pltpu.SemaphoreType.DMA,
        ],
    )
    def k(xl_hbm, xr_hbm, src_hbm, dst_hbm, att_hbm, bias_hbm, out_hbm,
          src_chunk, dst_chunk, csrc, cdst, xl_buf, xr_buf,
          att_buf, bias_buf, acc, psum, sem0, sem1):
        cid = lax.axis_index("c")
        sid = lax.axis_index("s")
        w = sid * 2 + cid

        i16 = lax.iota(jnp.int32, 16)
        zf16 = jnp.zeros((16,), jnp.float32)

        pltpu.sync_copy(att_hbm, att_buf)
        pltpu.sync_copy(bias_hbm, bias_buf)

        def pass_body(p, _):
            lo = p * (NW * R) + w * R
            hi = jnp.minimum(lo + R, N_NODES)

            def zacc(i, _):
                acc[pl.ds(i * 16, 16)] = zf16
                return 0
            lax.fori_loop(0, (R + 1) * HC // 16, zacc, 0)

            def zps(i, _):
                psum[pl.ds(i * 16, 16)] = zf16
                return 0
            lax.fori_loop(0, R + 1, zps, 0)

            def chunk_body(ck, _):
                eoff = ck * CH
                cpa = pltpu.async_copy(
                    src_hbm.at[pl.ds(eoff, CH)], src_chunk, sem0)
                cpb = pltpu.async_copy(
                    dst_hbm.at[pl.ds(eoff, CH)], dst_chunk, sem1)
                cpa.wait()
                cpb.wait()

                def compact(g, cur):
                    d16 = dst_chunk[pl.ds(g * 16, 16)]
                    s16 = src_chunk[pl.ds(g * 16, 16)]
                    m = (d16 >= lo) & (d16 < hi)
                    m32 = m.astype(jnp.int32)
                    cs = plsc.cumsum(m32)
                    pos = cur + cs - 1
                    plsc.store_scatter(cdst, [pos], d16 - lo, mask=m)
                    plsc.store_scatter(csrc, [pos], s16, mask=m)
                    return cur + cs[15]
                nsel = lax.fori_loop(0, GPC, compact, 0)

                plsc.store_scatter(cdst, [nsel + i16],
                                   jnp.full((16,), R, jnp.int32))
                plsc.store_scatter(csrc, [nsel + i16],
                                   jnp.zeros((16,), jnp.int32))
                ngroups = (nsel + 15) // 16

                def pg(g2, _):
                    o = g2 * 16
                    cs16 = csrc[pl.ds(o, 16)]
                    cd16 = cdst[pl.ds(o, 16)]
                    dg = jnp.minimum(cd16 + lo, N_NODES - 1)
                    cpc = pltpu.async_copy(xl_hbm.at[cs16], xl_buf, sem0)
                    cpd = pltpu.async_copy(xr_hbm.at[dg], xr_buf, sem1)
                    cpc.wait()
                    cpd.wait()

                    o16 = jnp.full((16,), o, jnp.int32)

                    def pedge(j, _):
                        djv = plsc.load_gather(cdst, [o16 + j])
                        dj = djv[0]
                        rb = dj * HC
                        pcon = zf16
                        for h in range(H):
                            xlw = []
                            a16 = zf16
                            for c in range(8):
                                off = h * C + c * 16
                                xv = xl_buf[j, pl.ds(off, 16)]
                                xlw.append(xv)
                                z = xv + xr_buf[j, pl.ds(off, 16)]
                                zl = jnp.maximum(z, NEG_SLOPE * z)
                                a16 = a16 + att_buf[pl.ds(off, 16)] * zl
                            a = plsc.cumsum(a16)[15]
                            pv = jnp.exp(jnp.full((16,), a, jnp.float32))
                            for c in range(8):
                                off = h * C + c * 16
                                plsc.addupdate(acc.at[pl.ds(rb + off, 16)],
                                               xlw[c] * pv)
                            pcon = pcon + jnp.where(i16 == h, pv, 0.0)
                        plsc.addupdate(psum.at[pl.ds(dj * 16, 16)], pcon)
                        return 0
                    lax.fori_loop(0, 16, pedge, 0)
                    return 0
                lax.fori_loop(0, ngroups, pg, 0)
                return 0
            lax.fori_loop(0, NCHUNKS, chunk_body, 0)

            def fin_blk(b, _):
                n0 = lo + b * 16

                @pl.when(n0 < N_NODES)
                def _():
                    def fin_row(j, _):
                        r = b * 16 + j
                        pv = psum[pl.ds(r * 16, 16)]
                        psum[pl.ds(r * 16, 16)] = 1.0 / (pv + 1e-16)
                        for h in range(H):
                            ib = plsc.load_gather(
                                psum,
                                [jnp.full((16,), r * 16 + h, jnp.int32)])
                            for c in range(8):
                                off = h * C + c * 16
                                xl_buf[j, pl.ds(off, 16)] = (
                                    acc[pl.ds(r * HC + off, 16)] * ib
                                    + bias_buf[pl.ds(off, 16)])
                        return 0
                    lax.fori_loop(0, 16, fin_row, 0)
                    pltpu.sync_copy(xl_buf, out_hbm.at[pl.ds(n0, 16)])
                return 0
            lax.fori_loop(0, R // 16, fin_blk, 0)
            return 0
        lax.fori_loop(0, PASSES, pass_body, 0)

    return k(xl, xr, src, dst, att1d, bias)


def kernel(x, edge_index, W_l, W_r, att, bias):
    src = edge_index[0].astype(jnp.int32)
    dst = edge_index[1].astype(jnp.int32)
    xl2, xr2 = _project(x, W_l, W_r)
    return _sc_gat(xl2, xr2, src, dst, att.reshape(-1),
                   bias.astype(jnp.float32))


# speedup vs baseline: 7.4797x; 1.7752x over previous
import functools

import jax
import jax.numpy as jnp
from jax import lax
from jax.experimental import pallas as pl
from jax.experimental.pallas import tpu as pltpu
from jax.experimental.pallas import tpu_sc as plsc

H = 8
C = 128
HC = H * C
NEG_SLOPE = 0.2

N_NODES = 10000
N_EDGES = 320000

NW = 32
R = 80
PASSES = 4
CR = 16 * R
EPT = N_EDGES // 16
CH = 2000
NCH1 = EPT // CH
CAP = NCH1 * (CH + 16) + CH
NSLOT = 32


def _mm_body(x_ref, wl_ref, wr_ref, xl_ref, xr_ref):
    x = x_ref[...]
    xl_ref[...] = jnp.dot(x, wl_ref[...], preferred_element_type=jnp.float32)
    xr_ref[...] = jnp.dot(x, wr_ref[...], preferred_element_type=jnp.float32)


def _project(x, W_l, W_r):
    N, d = x.shape
    BN = 400
    return pl.pallas_call(
        _mm_body,
        grid=(N // BN,),
        in_specs=[
            pl.BlockSpec((BN, d), lambda i: (i, 0)),
            pl.BlockSpec((d, HC), lambda i: (0, 0)),
            pl.BlockSpec((d, HC), lambda i: (0, 0)),
        ],
        out_specs=[
            pl.BlockSpec((BN, HC), lambda i: (i, 0)),
            pl.BlockSpec((BN, HC), lambda i: (i, 0)),
        ],
        out_shape=[
            jax.ShapeDtypeStruct((N, HC), jnp.float32),
            jax.ShapeDtypeStruct((N, HC), jnp.float32),
        ],
    )(x, W_l, W_r)


def _sc_gat(xl, xr, src, dst, att1d, bias):
    mesh = plsc.VectorSubcoreMesh(core_axis_name="c", subcore_axis_name="s")

    @functools.partial(
        pl.kernel,
        out_type=(jax.ShapeDtypeStruct((N_NODES, HC), jnp.float32),
                  jax.ShapeDtypeStruct((NSLOT * CAP,), jnp.int32),
                  jax.ShapeDtypeStruct((NSLOT * CAP,), jnp.int32),
                  jax.ShapeDtypeStruct((NSLOT * 16,), jnp.int32)),
        mesh=mesh,
        compiler_params=pltpu.CompilerParams(needs_layout_passes=False),
        scratch_types=[
            pltpu.VMEM((CH,), jnp.int32),
            pltpu.VMEM((CH,), jnp.int32),
            pltpu.VMEM((CH + 16,), jnp.int32),
            pltpu.VMEM((CH + 16,), jnp.int32),
            pltpu.VMEM((16, HC), jnp.float32),
            pltpu.VMEM((16, HC), jnp.float32),
            pltpu.VMEM((HC,), jnp.float32),
            pltpu.VMEM((HC,), jnp.float32),
            pltpu.VMEM(((R + 1) * HC,), jnp.float32),
            pltpu.VMEM(((R + 1) * 16,), jnp.float32),
            pltpu.VMEM((256,), jnp.int32),
            pltpu.VMEM((16,), jnp.int32),
            pltpu.SemaphoreType.DMA,
            pltpu.SemaphoreType.DMA,
        ],
    )
    def k(xl_hbm, xr_hbm, src_hbm, dst_hbm, att_hbm, bias_hbm,
          out_hbm, co_src, co_dst, counts,
          src_chunk, dst_chunk, csrc, cdst, xl_buf, xr_buf,
          att_buf, bias_buf, acc, psum, cnt_buf, cnt_stage,
          sem0, sem1):
        slot0 = lax.axis_index("c") * 16 * CAP
        cid = lax.axis_index("c")
        sid = lax.axis_index("s")

        i16 = lax.iota(jnp.int32, 16)
        zf16 = jnp.zeros((16,), jnp.float32)

        pltpu.sync_copy(att_hbm, att_buf)
        pltpu.sync_copy(bias_hbm, bias_buf)

        def pass_body(p, _):
            clo = p * (NW * R) + cid * CR
            chi = jnp.minimum(clo + CR, N_NODES)
            lo = clo + sid * R

            def zacc(i, _):
                acc[pl.ds(i * 16, 16)] = zf16
                return 0
            lax.fori_loop(0, (R + 1) * HC // 16, zacc, 0)

            def zps(i, _):
                psum[pl.ds(i * 16, 16)] = zf16
                return 0
            lax.fori_loop(0, R + 1, zps, 0)

            def ch1_body(ck, cc):
                eoff = sid * EPT + ck * CH
                cpa = pltpu.async_copy(
                    src_hbm.at[pl.ds(eoff, CH)], src_chunk, sem0)
                cpb = pltpu.async_copy(
                    dst_hbm.at[pl.ds(eoff, CH)], dst_chunk, sem1)
                cpa.wait()
                cpb.wait()

                def compact(g, cur):
                    d16 = dst_chunk[pl.ds(g * 16, 16)]
                    s16 = src_chunk[pl.ds(g * 16, 16)]
                    m = (d16 >= clo) & (d16 < chi)
                    m32 = m.astype(jnp.int32)
                    cs = plsc.cumsum(m32)
                    pos = cur + cs - 1
                    plsc.store_scatter(cdst, [pos], d16 - clo, mask=m)
                    plsc.store_scatter(csrc, [pos], s16, mask=m)
                    return cur + cs[15]
                nsel = lax.fori_loop(0, CH // 16, compact, 0)

                plsc.store_scatter(cdst, [nsel + i16],
                                   jnp.full((16,), CR, jnp.int32))
                plsc.store_scatter(csrc, [nsel + i16],
                                   jnp.zeros((16,), jnp.int32))
                ccm = pl.multiple_of(cc, 16)
                pltpu.sync_copy(
                    csrc, co_src.at[pl.ds(slot0 + sid * CAP + ccm, CH + 16)])
                pltpu.sync_copy(
                    cdst, co_dst.at[pl.ds(slot0 + sid * CAP + ccm, CH + 16)])
                return cc + ((nsel + 15) // 16) * 16
            total = lax.fori_loop(0, NCH1, ch1_body, 0)

            cnt_stage[pl.ds(0, 16)] = jnp.full((16,), 1, jnp.int32) * total
            pltpu.sync_copy(cnt_stage,
                counts.at[pl.ds((cid * 16 + sid) * 16, 16)])
            plsc.subcore_barrier()

            pltpu.sync_copy(counts.at[pl.ds(cid * 16 * 16, 256)], cnt_buf)

            def u_body(u, _):
                cntu = cnt_buf[pl.ds(u * 16, 16)][0]
                nq = (cntu + CH - 1) // CH

                def ch2_body(q, _):
                    qoff = q * CH
                    cpa = pltpu.async_copy(
                        co_src.at[pl.ds(slot0 + u * CAP + qoff, CH)],
                        src_chunk, sem0)
                    cpb = pltpu.async_copy(
                        co_dst.at[pl.ds(slot0 + u * CAP + qoff, CH)],
                        dst_chunk, sem1)
                    cpa.wait()
                    cpb.wait()
                    gq = (jnp.minimum(cntu - qoff, CH) + 15) // 16

                    def compact(g, cur):
                        d16 = dst_chunk[pl.ds(g * 16, 16)]
                        s16 = src_chunk[pl.ds(g * 16, 16)]
                        m = ((d16 >= sid * R) & (d16 < sid * R + R)
                             & (qoff + g * 16 + i16 < cntu))
                        m32 = m.astype(jnp.int32)
                        cs = plsc.cumsum(m32)
                        pos = cur + cs - 1
                        plsc.store_scatter(cdst, [pos], d16 - sid * R, mask=m)
                        plsc.store_scatter(csrc, [pos], s16, mask=m)
                        return cur + cs[15]
                    nsel = lax.fori_loop(0, gq, compact, 0)

                    plsc.store_scatter(cdst, [nsel + i16],
                                       jnp.full((16,), R, jnp.int32))
                    plsc.store_scatter(csrc, [nsel + i16],
                                       jnp.zeros((16,), jnp.int32))
                    ngroups = (nsel + 15) // 16

                    def pg(g2, _):
                        o = g2 * 16
                        cs16 = csrc[pl.ds(o, 16)]
                        cd16 = cdst[pl.ds(o, 16)]
                        dg = jnp.minimum(cd16 + lo, N_NODES - 1)
                        cpc = pltpu.async_copy(xl_hbm.at[cs16], xl_buf, sem0)
                        cpd = pltpu.async_copy(xr_hbm.at[dg], xr_buf, sem1)
                        cpc.wait()
                        cpd.wait()

                        o16 = jnp.full((16,), o, jnp.int32)

                        def pedge(j, _):
                            djv = plsc.load_gather(cdst, [o16 + j])
                            dj = djv[0]
                            rb = dj * HC
                            pcon = zf16
                            for h in range(H):
                                xlw = []
                                a16 = zf16
                                for c in range(8):
                                    off = h * C + c * 16
                                    xv = xl_buf[j, pl.ds(off, 16)]
                                    xlw.append(xv)
                                    z = xv + xr_buf[j, pl.ds(off, 16)]
                                    zl = jnp.maximum(z, NEG_SLOPE * z)
                                    a16 = a16 + att_buf[pl.ds(off, 16)] * zl
                                a = plsc.cumsum(a16)[15]
                                pv = jnp.exp(jnp.full((16,), a, jnp.float32))
                                for c in range(8):
                                    off = h * C + c * 16
                                    plsc.addupdate(
                                        acc.at[pl.ds(rb + off, 16)],
                                        xlw[c] * pv)
                                pcon = pcon + jnp.where(i16 == h, pv, 0.0)
                            plsc.addupdate(psum.at[pl.ds(dj * 16, 16)], pcon)
                            return 0
                        lax.fori_loop(0, 16, pedge, 0)
                        return 0
                    lax.fori_loop(0, ngroups, pg, 0)
                    return 0
                lax.fori_loop(0, nq, ch2_body, 0)
                return 0
            lax.fori_loop(0, 16, u_body, 0)

            def fin_blk(b, _):
                n0 = lo + b * 16

                @pl.when(n0 < N_NODES)
                def _():
                    def fin_row(j, _):
                        r = b * 16 + j
                        pv = psum[pl.ds(r * 16, 16)]
                        psum[pl.ds(r * 16, 16)] = 1.0 / (pv + 1e-16)
                        for h in range(H):
                            ib = plsc.load_gather(
                                psum,
                                [jnp.full((16,), r * 16 + h, jnp.int32)])
                            for c in range(8):
                                off = h * C + c * 16
                                xl_buf[j, pl.ds(off, 16)] = (
                                    acc[pl.ds(r * HC + off, 16)] * ib
                                    + bias_buf[pl.ds(off, 16)])
                        return 0
                    lax.fori_loop(0, 16, fin_row, 0)
                    pltpu.sync_copy(xl_buf, out_hbm.at[pl.ds(n0, 16)])
                return 0
            lax.fori_loop(0, R // 16, fin_blk, 0)
            plsc.subcore_barrier()
            return 0
        lax.fori_loop(0, PASSES, pass_body, 0)

    return k(xl, xr, src, dst, att1d, bias)[0]


def kernel(x, edge_index, W_l, W_r, att, bias):
    src = edge_index[0].astype(jnp.int32)
    dst = edge_index[1].astype(jnp.int32)
    xl2, xr2 = _project(x, W_l, W_r)
    return _sc_gat(xl2, xr2, src, dst, att.reshape(-1),
                   bias.astype(jnp.float32))


# speedup vs baseline: 7.9760x; 1.0664x over previous
import functools

import jax
import jax.numpy as jnp
from jax import lax
from jax.experimental import pallas as pl
from jax.experimental.pallas import tpu as pltpu
from jax.experimental.pallas import tpu_sc as plsc

H = 8
C = 128
HC = H * C
NEG_SLOPE = 0.2

N_NODES = 10000
N_EDGES = 320000

NW = 32
R = 48
PASSES = 7
CR = 16 * R
EPT = N_EDGES // 16
CH = 2000
NCH1 = EPT // CH
CAP = NCH1 * (CH + 16) + CH
NSLOT = 32


def _mm_body(x_ref, wl_ref, wr_ref, xl_ref, xr_ref):
    x = x_ref[...]
    xl_ref[...] = jnp.dot(x, wl_ref[...], preferred_element_type=jnp.float32)
    xr_ref[...] = jnp.dot(x, wr_ref[...], preferred_element_type=jnp.float32)


def _project(x, W_l, W_r):
    N, d = x.shape
    BN = 400
    return pl.pallas_call(
        _mm_body,
        grid=(N // BN,),
        in_specs=[
            pl.BlockSpec((BN, d), lambda i: (i, 0)),
            pl.BlockSpec((d, HC), lambda i: (0, 0)),
            pl.BlockSpec((d, HC), lambda i: (0, 0)),
        ],
        out_specs=[
            pl.BlockSpec((BN, HC), lambda i: (i, 0)),
            pl.BlockSpec((BN, HC), lambda i: (i, 0)),
        ],
        out_shape=[
            jax.ShapeDtypeStruct((N, HC), jnp.float32),
            jax.ShapeDtypeStruct((N, HC), jnp.float32),
        ],
    )(x, W_l, W_r)


def _sc_gat(xl, xr, src, dst, att1d, bias):
    mesh = plsc.VectorSubcoreMesh(core_axis_name="c", subcore_axis_name="s")

    @functools.partial(
        pl.kernel,
        out_type=(jax.ShapeDtypeStruct((N_NODES, HC), jnp.float32),
                  jax.ShapeDtypeStruct((NSLOT * CAP,), jnp.int32),
                  jax.ShapeDtypeStruct((NSLOT * CAP,), jnp.int32),
                  jax.ShapeDtypeStruct((NSLOT * 16,), jnp.int32)),
        mesh=mesh,
        compiler_params=pltpu.CompilerParams(needs_layout_passes=False),
        scratch_types=[
            pltpu.VMEM((CH,), jnp.int32),
            pltpu.VMEM((CH,), jnp.int32),
            pltpu.VMEM((CH + 32,), jnp.int32),
            pltpu.VMEM((CH + 32,), jnp.int32),
            pltpu.VMEM((16, HC), jnp.float32),
            pltpu.VMEM((16, HC), jnp.float32),
            pltpu.VMEM((16, HC), jnp.float32),
            pltpu.VMEM((16, HC), jnp.float32),
            pltpu.VMEM((HC,), jnp.float32),
            pltpu.VMEM((HC,), jnp.float32),
            pltpu.VMEM(((R + 1) * HC,), jnp.float32),
            pltpu.VMEM(((R + 1) * 16,), jnp.float32),
            pltpu.VMEM((256,), jnp.int32),
            pltpu.VMEM((16,), jnp.int32),
            pltpu.SemaphoreType.DMA,
            pltpu.SemaphoreType.DMA,
            pltpu.SemaphoreType.DMA,
            pltpu.SemaphoreType.DMA,
        ],
    )
    def k(xl_hbm, xr_hbm, src_hbm, dst_hbm, att_hbm, bias_hbm,
          out_hbm, co_src, co_dst, counts,
          src_chunk, dst_chunk, csrc, cdst, xl_a, xr_a, xl_b, xr_b,
          att_buf, bias_buf, acc, psum, cnt_buf, cnt_stage,
          sem0, sem1, sem2, sem3):
        slot0 = lax.axis_index("c") * 16 * CAP
        cid = lax.axis_index("c")
        sid = lax.axis_index("s")

        i16 = lax.iota(jnp.int32, 16)
        zf16 = jnp.zeros((16,), jnp.float32)

        pltpu.sync_copy(att_hbm, att_buf)
        pltpu.sync_copy(bias_hbm, bias_buf)

        def pass_body(p, _):
            clo = p * (NW * R) + cid * CR
            chi = jnp.minimum(clo + CR, N_NODES)
            lo = clo + sid * R

            def zacc(i, _):
                acc[pl.ds(i * 16, 16)] = zf16
                return 0
            lax.fori_loop(0, (R + 1) * HC // 16, zacc, 0)

            def zps(i, _):
                psum[pl.ds(i * 16, 16)] = zf16
                return 0
            lax.fori_loop(0, R + 1, zps, 0)

            def ch1_body(ck, cc):
                eoff = sid * EPT + ck * CH
                cpa = pltpu.async_copy(
                    src_hbm.at[pl.ds(eoff, CH)], src_chunk, sem0)
                cpb = pltpu.async_copy(
                    dst_hbm.at[pl.ds(eoff, CH)], dst_chunk, sem1)
                cpa.wait()
                cpb.wait()

                def compact(g, cur):
                    d16 = dst_chunk[pl.ds(g * 16, 16)]
                    s16 = src_chunk[pl.ds(g * 16, 16)]
                    m = (d16 >= clo) & (d16 < chi)
                    m32 = m.astype(jnp.int32)
                    cs = plsc.cumsum(m32)
                    pos = cur + cs - 1
                    plsc.store_scatter(cdst, [pos], d16 - clo, mask=m)
                    plsc.store_scatter(csrc, [pos], s16, mask=m)
                    return cur + cs[15]
                nsel = lax.fori_loop(0, CH // 16, compact, 0)

                plsc.store_scatter(cdst, [nsel + i16],
                                   jnp.full((16,), CR, jnp.int32))
                plsc.store_scatter(csrc, [nsel + i16],
                                   jnp.zeros((16,), jnp.int32))
                ccm = pl.multiple_of(cc, 16)
                pltpu.sync_copy(
                    csrc.at[pl.ds(0, CH + 16)],
                    co_src.at[pl.ds(slot0 + sid * CAP + ccm, CH + 16)])
                pltpu.sync_copy(
                    cdst.at[pl.ds(0, CH + 16)],
                    co_dst.at[pl.ds(slot0 + sid * CAP + ccm, CH + 16)])
                return cc + ((nsel + 15) // 16) * 16
            total = lax.fori_loop(0, NCH1, ch1_body, 0)

            cnt_stage[pl.ds(0, 16)] = jnp.full((16,), 1, jnp.int32) * total
            pltpu.sync_copy(cnt_stage,
                counts.at[pl.ds((cid * 16 + sid) * 16, 16)])
            plsc.subcore_barrier()

            pltpu.sync_copy(counts.at[pl.ds(cid * 16 * 16, 256)], cnt_buf)

            def u_body(u, _):
                cntu = cnt_buf[pl.ds(u * 16, 16)][0]
                nq = (cntu + CH - 1) // CH

                def ch2_body(q, _):
                    qoff = q * CH
                    cpa = pltpu.async_copy(
                        co_src.at[pl.ds(slot0 + u * CAP + qoff, CH)],
                        src_chunk, sem0)
                    cpb = pltpu.async_copy(
                        co_dst.at[pl.ds(slot0 + u * CAP + qoff, CH)],
                        dst_chunk, sem1)
                    cpa.wait()
                    cpb.wait()
                    gq = (jnp.minimum(cntu - qoff, CH) + 15) // 16

                    def compact(g, cur):
                        d16 = dst_chunk[pl.ds(g * 16, 16)]
                        s16 = src_chunk[pl.ds(g * 16, 16)]
                        m = ((d16 >= sid * R) & (d16 < sid * R + R)
                             & (qoff + g * 16 + i16 < cntu))
                        m32 = m.astype(jnp.int32)
                        cs = plsc.cumsum(m32)
                        pos = cur + cs - 1
                        plsc.store_scatter(cdst, [pos], d16 - sid * R, mask=m)
                        plsc.store_scatter(csrc, [pos], s16, mask=m)
                        return cur + cs[15]
                    nsel = lax.fori_loop(0, gq, compact, 0)

                    plsc.store_scatter(cdst, [nsel + i16],
                                       jnp.full((16,), R, jnp.int32))
                    plsc.store_scatter(csrc, [nsel + i16],
                                       jnp.zeros((16,), jnp.int32))
                    plsc.store_scatter(cdst, [nsel + 16 + i16],
                                       jnp.full((16,), R, jnp.int32))
                    plsc.store_scatter(csrc, [nsel + 16 + i16],
                                       jnp.zeros((16,), jnp.int32))
                    npairs = (nsel + 31) // 32

                    def issue(o, xbuf, rbuf, sa, sb):
                        cs16 = csrc[pl.ds(o, 16)]
                        cd16 = cdst[pl.ds(o, 16)]
                        dg = jnp.minimum(cd16 + lo, N_NODES - 1)
                        pltpu.async_copy(xl_hbm.at[cs16], xbuf, sa)
                        pltpu.async_copy(xr_hbm.at[dg], rbuf, sb)

                    def drain(xbuf, rbuf, sa, sb):
                        pltpu.make_async_copy(
                            xl_hbm.at[pl.ds(0, 16)], xbuf, sa).wait()
                        pltpu.make_async_copy(
                            xr_hbm.at[pl.ds(0, 16)], rbuf, sb).wait()

                    def pedge_grp(o16, xbuf, rbuf):
                        def pedge(j, _):
                            djv = plsc.load_gather(cdst, [o16 + j])
                            dj = djv[0]
                            rb = dj * HC
                            pcon = zf16
                            for h in range(H):
                                xlw = []
                                a16 = zf16
                                for c in range(8):
                                    off = h * C + c * 16
                                    xv = xbuf[j, pl.ds(off, 16)]
                                    xlw.append(xv)
                                    z = xv + rbuf[j, pl.ds(off, 16)]
                                    zl = jnp.maximum(z, NEG_SLOPE * z)
                                    a16 = a16 + att_buf[pl.ds(off, 16)] * zl
                                a = plsc.cumsum(a16)[15]
                                pv = jnp.exp(jnp.full((16,), a, jnp.float32))
                                for c in range(8):
                                    off = h * C + c * 16
                                    plsc.addupdate(
                                        acc.at[pl.ds(rb + off, 16)],
                                        xlw[c] * pv)
                                pcon = pcon + jnp.where(i16 == h, pv, 0.0)
                            plsc.addupdate(psum.at[pl.ds(dj * 16, 16)], pcon)
                            return 0
                        lax.fori_loop(0, 16, pedge, 0)

                    @pl.when(npairs > 0)
                    def _():
                        issue(0, xl_a, xr_a, sem0, sem1)

                    def pg(gp, _):
                        o = gp * 32
                        issue(o + 16, xl_b, xr_b, sem2, sem3)
                        drain(xl_a, xr_a, sem0, sem1)
                        pedge_grp(jnp.full((16,), o, jnp.int32), xl_a, xr_a)

                        @pl.when(gp + 1 < npairs)
                        def _():
                            issue(o + 32, xl_a, xr_a, sem0, sem1)
                        drain(xl_b, xr_b, sem2, sem3)
                        pedge_grp(jnp.full((16,), o + 16, jnp.int32),
                                  xl_b, xr_b)
                        return 0
                    lax.fori_loop(0, npairs, pg, 0)
                    return 0
                lax.fori_loop(0, nq, ch2_body, 0)
                return 0
            lax.fori_loop(0, 16, u_body, 0)

            def fin_blk(b, _):
                n0 = lo + b * 16

                @pl.when(n0 < N_NODES)
                def _():
                    def fin_row(j, _):
                        r = b * 16 + j
                        pv = psum[pl.ds(r * 16, 16)]
                        psum[pl.ds(r * 16, 16)] = 1.0 / (pv + 1e-16)
                        for h in range(H):
                            ib = plsc.load_gather(
                                psum,
                                [jnp.full((16,), r * 16 + h, jnp.int32)])
                            for c in range(8):
                                off = h * C + c * 16
                                xl_a[j, pl.ds(off, 16)] = (
                                    acc[pl.ds(r * HC + off, 16)] * ib
                                    + bias_buf[pl.ds(off, 16)])
                        return 0
                    lax.fori_loop(0, 16, fin_row, 0)
                    pltpu.sync_copy(xl_a, out_hbm.at[pl.ds(n0, 16)])
                return 0
            lax.fori_loop(0, R // 16, fin_blk, 0)
            plsc.subcore_barrier()
            return 0
        lax.fori_loop(0, PASSES, pass_body, 0)

    return k(xl, xr, src, dst, att1d, bias)[0]


def kernel(x, edge_index, W_l, W_r, att, bias):
    src = edge_index[0].astype(jnp.int32)
    dst = edge_index[1].astype(jnp.int32)
    xl2, xr2 = _project(x, W_l, W_r)
    return _sc_gat(xl2, xr2, src, dst, att.reshape(-1),
                   bias.astype(jnp.float32))
